# baseline (speedup 1.0000x reference)
import jax
import jax.numpy as jnp
from jax import lax
from jax.experimental import pallas as pl
from jax.experimental.pallas import tpu as pltpu
from jax.experimental.pallas import tpu_sc as plsc

N_NODES = 10000
N_EDGES = 320000
D = 128

NC, NS = 2, 16
NW = NC * NS
CHUNK = 128
G = 80
EPT = G * CHUNK
E_PAD = NW * EPT

ACC_ROWS = 10240
RPT = ACC_ROWS // NS


def _mm_body(x_ref, w_ref, o_ref):
    o_ref[...] = lax.dot_general(
        x_ref[...], w_ref[...], (((1,), (1,)), ((), ())),
        preferred_element_type=jnp.float32)


def _edge_transform(x, W_edge):
    blk = 1000
    return pl.pallas_call(
        _mm_body,
        grid=(N_NODES // blk,),
        in_specs=[pl.BlockSpec((blk, D), lambda i: (i, 0)),
                  pl.BlockSpec((D, D), lambda i: (0, 0))],
        out_specs=pl.BlockSpec((blk, D), lambda i: (i, 0)),
        out_shape=jax.ShapeDtypeStruct((N_NODES, D), jnp.float32),
    )(x, W_edge)


def _sc_body(idx_hbm, y_hbm, acc_out, cnt_out,
             idx0, idx1, rows0, rows1, ones_v, zc_v,
             acc_sh, cnt_sh, semg0, semg1, sems0, sems1, semc0, semc1):
    c = lax.axis_index("c")
    s = lax.axis_index("s")
    wid = s * NC + c

    idx_b = (idx0, idx1)
    rows_b = (rows0, rows1)
    semg = (semg0, semg1)
    sems = (sems0, sems1)
    semc = (semc0, semc1)

    zeros16 = jnp.zeros((16,), jnp.float32)
    ones16 = zeros16 + 1.0

    def _zrow(i, _):
        def _zcol(j, _):
            rows0[i, pl.ds(j * 16, 16)] = zeros16
            return 0
        return lax.fori_loop(0, D // 16, _zcol, 0)
    lax.fori_loop(0, CHUNK, _zrow, 0)

    for j in range(CHUNK // 16):
        ones_v[pl.ds(j * 16, 16)] = ones16

    def _zc(i, _):
        zc_v[pl.ds(i * 16, 16)] = zeros16
        return 0
    lax.fori_loop(0, RPT // 16, _zc, 0)

    base_r = s * RPT
    for k in range(RPT // CHUNK):
        pltpu.sync_copy(rows0, acc_sh.at[pl.ds(base_r + k * CHUNK, CHUNK)])
    pltpu.sync_copy(zc_v, cnt_sh.at[pl.ds(base_r, RPT)])

    ebase = wid * G * (2 * CHUNK)

    def _load_idx(g, b):
        pltpu.sync_copy(idx_hbm.at[pl.ds(ebase + g * (2 * CHUNK), 2 * CHUNK)],
                        idx_b[b])

    def _gather(g, b):
        return pltpu.async_copy(
            y_hbm.at[idx_b[b].at[pl.ds(0, CHUNK)]], rows_b[b], sem=semg[b])

    def _scatter(b):
        return pltpu.async_copy(
            rows_b[b], acc_sh.at[idx_b[b].at[pl.ds(CHUNK, CHUNK)]],
            add=True, sem=sems[b])

    def _count(b):
        return pltpu.async_copy(
            ones_v, cnt_sh.at[idx_b[b].at[pl.ds(CHUNK, CHUNK)]],
            add=True, sem=semc[b])

    _load_idx(0, 0)
    gc = {0: _gather(0, 0)}
    sc = {}
    cc = {}

    plsc.subcore_barrier()

    for g in range(G):
        b = g & 1
        nb = b ^ 1
        if g + 1 < G:
            if g >= 1:
                sc[g - 1].wait()
                cc[g - 1].wait()
            _load_idx(g + 1, nb)
            gc[g + 1] = _gather(g + 1, nb)
        gc[g].wait()
        sc[g] = _scatter(b)
        cc[g] = _count(b)

    sc[G - 2].wait()
    cc[G - 2].wait()
    sc[G - 1].wait()
    cc[G - 1].wait()

    plsc.subcore_barrier()

    pltpu.sync_copy(acc_sh.at[pl.ds(base_r, RPT)],
                    acc_out.at[c, pl.ds(base_r, RPT)])
    pltpu.sync_copy(cnt_sh.at[pl.ds(base_r, RPT)],
                    cnt_out.at[c, pl.ds(base_r, RPT)])


def _sc_scatter(idx_packed, y):
    mesh = plsc.VectorSubcoreMesh(core_axis_name="c", subcore_axis_name="s")
    return pl.kernel(
        _sc_body,
        out_type=(jax.ShapeDtypeStruct((NC, ACC_ROWS, D), jnp.float32),
                  jax.ShapeDtypeStruct((NC, ACC_ROWS), jnp.float32)),
        mesh=mesh,
        scratch_types=[
            pltpu.VMEM((2 * CHUNK,), jnp.int32),
            pltpu.VMEM((2 * CHUNK,), jnp.int32),
            pltpu.VMEM((CHUNK, D), jnp.float32),
            pltpu.VMEM((CHUNK, D), jnp.float32),
            pltpu.VMEM((CHUNK,), jnp.float32),
            pltpu.VMEM((RPT,), jnp.float32),
            pltpu.VMEM_SHARED((ACC_ROWS, D), jnp.float32),
            pltpu.VMEM_SHARED((ACC_ROWS,), jnp.float32),
            pltpu.SemaphoreType.DMA,
            pltpu.SemaphoreType.DMA,
            pltpu.SemaphoreType.DMA,
            pltpu.SemaphoreType.DMA,
            pltpu.SemaphoreType.DMA,
            pltpu.SemaphoreType.DMA,
        ],
    )(idx_packed, y)


def _fin_body(x_ref, w_ref, b_ref, acc_ref, cnt_ref, o_ref):
    z = lax.dot_general(
        x_ref[...], w_ref[...], (((1,), (1,)), ((), ())),
        preferred_element_type=jnp.float32) + b_ref[...]
    a = acc_ref[0] + acc_ref[1]
    cnt = cnt_ref[0] + cnt_ref[1]
    h1 = a / jnp.maximum(cnt, 1.0)
    o_ref[...] = jnp.maximum(z + h1, 0.0)


def _finalize(x, W_self, b2, acc, cnt):
    blk = 1000
    return pl.pallas_call(
        _fin_body,
        grid=(N_NODES // blk,),
        in_specs=[pl.BlockSpec((blk, D), lambda i: (i, 0)),
                  pl.BlockSpec((D, D), lambda i: (0, 0)),
                  pl.BlockSpec((1, D), lambda i: (0, 0)),
                  pl.BlockSpec((NC, blk, D), lambda i: (0, i, 0)),
                  pl.BlockSpec((NC, blk, 1), lambda i: (0, i, 0))],
        out_specs=pl.BlockSpec((blk, D), lambda i: (i, 0)),
        out_shape=jax.ShapeDtypeStruct((N_NODES, D), jnp.float32),
    )(x, W_self, b2, acc, cnt)


def kernel(x, edge_index, W_edge, W_self, b_self):
    src = edge_index[0]
    dst = edge_index[1]
    pad = E_PAD - N_EDGES
    src_p = jnp.concatenate([src, jnp.zeros((pad,), jnp.int32)])
    dst_p = jnp.concatenate([dst, jnp.full((pad,), N_NODES, jnp.int32)])
    idx_packed = jnp.stack(
        [src_p.reshape(NW * G, CHUNK), dst_p.reshape(NW * G, CHUNK)],
        axis=1).reshape(-1)
    y = _edge_transform(x, W_edge)
    acc, cnt = _sc_scatter(idx_packed, y)
    cnt_col = cnt[:, :N_NODES, None]
    return _finalize(x, W_self, b_self[None, :], acc, cnt_col)


# speedup vs baseline: 1.3151x; 1.3151x over previous
import jax
import jax.numpy as jnp
from jax import lax
from jax.experimental import pallas as pl
from jax.experimental.pallas import tpu as pltpu
from jax.experimental.pallas import tpu_sc as plsc

N_NODES = 10000
N_EDGES = 320000
D = 128

NC, NS = 2, 16
NW = NC * NS
CHUNK = 128
G = -(-N_EDGES // (NW * CHUNK))
EPT = G * CHUNK
E_PAD = NW * EPT
Y_PAD = 10240
YPT = Y_PAD // NS

ACC_ROWS = 10240
RPT = ACC_ROWS // NS


def _mm_body(x_ref, w_ref, o_ref):
    o_ref[...] = lax.dot_general(
        x_ref[...], w_ref[...], (((1,), (1,)), ((), ())),
        preferred_element_type=jnp.float32)


def _edge_transform(x, W_edge):
    blk = 1000
    return pl.pallas_call(
        _mm_body,
        grid=(N_NODES // blk,),
        in_specs=[pl.BlockSpec((blk, D), lambda i: (i, 0)),
                  pl.BlockSpec((D, D), lambda i: (0, 0))],
        out_specs=pl.BlockSpec((blk, D), lambda i: (i, 0)),
        out_shape=jax.ShapeDtypeStruct((N_NODES, D), jnp.float32),
    )(x, W_edge)


def _sc_body(idx_hbm, y_hbm, acc_out, cnt_out,
             idx_all, rows_v, ones_v, zc_v, acc_sh, cnt_sh, sem):
    c = lax.axis_index("c")
    s = lax.axis_index("s")
    wid = s * NC + c

    zeros16 = jnp.zeros((16,), jnp.float32)
    zeros16f = jnp.zeros((16,), jnp.float32)
    ones16f = zeros16f + 1.0

    pltpu.sync_copy(idx_hbm.at[pl.ds(wid * 2 * EPT, 2 * EPT)], idx_all)

    for i in range(CHUNK):
        for j in range(D // 16):
            rows_v[i, pl.ds(j * 16, 16)] = zeros16

    for j in range(CHUNK // 16):
        ones_v[pl.ds(j * 16, 16)] = ones16f

    def _zc(i, _):
        zc_v[pl.ds(i * 16, 16)] = zeros16f
        return 0
    lax.fori_loop(0, RPT // 16, _zc, 0)

    base_r = pl.multiple_of(s * RPT, 16)
    for k in range(RPT // CHUNK):
        pltpu.sync_copy(rows_v, acc_sh.at[pl.ds(base_r + k * CHUNK, CHUNK)])
    pltpu.sync_copy(zc_v, cnt_sh.at[pl.ds(base_r, RPT)])

    plsc.subcore_barrier()

    def _chunk(g, _):
        b = g * CHUNK
        pltpu.async_copy(y_hbm.at[idx_all.at[pl.ds(b, CHUNK)]],
                         rows_v, sem).wait()
        pltpu.sync_copy(rows_v,
                        acc_sh.at[idx_all.at[pl.ds(EPT + b, CHUNK)]],
                        add=True)
        pltpu.sync_copy(ones_v,
                        cnt_sh.at[idx_all.at[pl.ds(EPT + b, CHUNK)]],
                        add=True)
        return 0
    lax.fori_loop(0, G, _chunk, 0)

    plsc.subcore_barrier()

    pltpu.sync_copy(acc_sh.at[pl.ds(base_r, RPT)],
                    acc_out.at[c, pl.ds(base_r, RPT)])
    pltpu.sync_copy(cnt_sh.at[pl.ds(base_r, RPT)],
                    cnt_out.at[c, pl.ds(base_r, RPT)])


def _sc_scatter(idx_packed, y):
    mesh = plsc.VectorSubcoreMesh(core_axis_name="c", subcore_axis_name="s")
    return pl.kernel(
        _sc_body,
        out_type=(jax.ShapeDtypeStruct((NC, ACC_ROWS, D), jnp.float32),
                  jax.ShapeDtypeStruct((NC, ACC_ROWS), jnp.float32)),
        mesh=mesh,
        scratch_types=[
            pltpu.VMEM((2 * EPT,), jnp.int32),
            pltpu.VMEM((CHUNK, D), jnp.float32),
            pltpu.VMEM((CHUNK,), jnp.float32),
            pltpu.VMEM((RPT,), jnp.float32),
            pltpu.VMEM_SHARED((ACC_ROWS, D), jnp.float32),
            pltpu.VMEM_SHARED((ACC_ROWS,), jnp.float32),
            pltpu.SemaphoreType.DMA,
        ],
    )(idx_packed, y)


def _fin_body(x_ref, w_ref, b_ref, acc_ref, cnt_ref, o_ref):
    z = lax.dot_general(
        x_ref[...], w_ref[...], (((1,), (1,)), ((), ())),
        preferred_element_type=jnp.float32) + b_ref[...]
    a = acc_ref[0] + acc_ref[1]
    cnt = cnt_ref[0] + cnt_ref[1]
    h1 = a / jnp.maximum(cnt, 1.0)
    o_ref[...] = jnp.maximum(z + h1, 0.0)


def _finalize(x, W_self, b2, acc, cnt):
    blk = 1000
    return pl.pallas_call(
        _fin_body,
        grid=(N_NODES // blk,),
        in_specs=[pl.BlockSpec((blk, D), lambda i: (i, 0)),
                  pl.BlockSpec((D, D), lambda i: (0, 0)),
                  pl.BlockSpec((1, D), lambda i: (0, 0)),
                  pl.BlockSpec((NC, blk, D), lambda i: (0, i, 0)),
                  pl.BlockSpec((NC, blk, 1), lambda i: (0, i, 0))],
        out_specs=pl.BlockSpec((blk, D), lambda i: (i, 0)),
        out_shape=jax.ShapeDtypeStruct((N_NODES, D), jnp.float32),
    )(x, W_self, b2, acc, cnt)


def kernel(x, edge_index, W_edge, W_self, b_self):
    src = edge_index[0]
    dst = edge_index[1]
    pad = E_PAD - N_EDGES
    src_p = jnp.concatenate([src, jnp.zeros((pad,), jnp.int32)])
    dst_p = jnp.concatenate([dst, jnp.full((pad,), N_NODES, jnp.int32)])
    idx_packed = jnp.concatenate(
        [src_p.reshape(NW, EPT), dst_p.reshape(NW, EPT)], axis=1).reshape(-1)
    y = _edge_transform(x, W_edge)
    acc, cnt = _sc_scatter(idx_packed, y)
    cnt_col = cnt[:, :N_NODES, None]
    return _finalize(x, W_self, b_self[None, :], acc, cnt_col)
